# baseline (speedup 1.0000x reference)
import functools

import jax
import jax.numpy as jnp
from jax import lax
from jax.experimental import pallas as pl
from jax.experimental.pallas import tpu as pltpu
from jax.experimental.pallas import tpu_sc as plsc

_VOCAB = 100000
_EMB = 128
_BATCH = 16384
_HALF = _BATCH // 2
_NEG = 20
_LANES = 16


def _sc_gather(x_idx, y_idx, neg_idx, w1, w2, base_off, batch):
    info = plsc.get_sparse_core_info()
    nc, ns = info.num_cores, info.num_subcores
    nw = nc * ns
    bpw = batch // nw
    nbuf = 4
    ch = bpw // nbuf

    out_type = [
        jax.ShapeDtypeStruct((batch, _EMB), jnp.float32),
        jax.ShapeDtypeStruct((batch, _LANES), jnp.float32),
    ]
    with_neg = neg_idx is not None
    if with_neg:
        out_type.append(jax.ShapeDtypeStruct((_NEG, _EMB), jnp.float32))

    scratch = [
        pltpu.VMEM((bpw,), jnp.int32),
        pltpu.VMEM((bpw,), jnp.int32),
        [pltpu.VMEM((ch, _EMB), jnp.float32)] * nbuf,
        [pltpu.VMEM((ch, _EMB), jnp.float32)] * nbuf,
        pltpu.VMEM((bpw, _LANES), jnp.float32),
        pltpu.SemaphoreType.DMA,
        pltpu.SemaphoreType.DMA,
        [pltpu.SemaphoreType.DMA] * nbuf,
        pltpu.SemaphoreType.DMA,
        pltpu.SemaphoreType.DMA,
    ]
    if with_neg:
        scratch += [pltpu.VMEM((_NEG,), jnp.int32),
                    pltpu.VMEM((_NEG, _EMB), jnp.float32)]

    @functools.partial(
        pl.kernel,
        out_type=tuple(out_type),
        mesh=plsc.VectorSubcoreMesh(core_axis_name="c", subcore_axis_name="s"),
        scratch_types=scratch,
    )
    def gather_kernel(*refs):
        if with_neg:
            (xi, yi, ni, w1h, w2h, xo, po, no,
             idx_x, idx_y, xbufs, ybufs, pos_v,
             gxsem, gysem, wsems, psem, isem, nidx_v, nrows_v) = refs
        else:
            (xi, yi, w1h, w2h, xo, po,
             idx_x, idx_y, xbufs, ybufs, pos_v,
             gxsem, gysem, wsems, psem, isem) = refs
        wid = lax.axis_index("s") * nc + lax.axis_index("c")
        base = wid * bpw
        ix = pltpu.async_copy(xi.at[pl.ds(base_off + base, bpw)], idx_x, isem)
        iy = pltpu.async_copy(yi.at[pl.ds(base_off + base, bpw)], idx_y, isem)
        ix.wait()
        gx = [pltpu.async_copy(w1h.at[idx_x.at[pl.ds(j * ch, ch)]],
                               xbufs[j], gxsem) for j in range(nbuf)]
        iy.wait()
        gy = [pltpu.async_copy(w2h.at[idx_y.at[pl.ds(j * ch, ch)]],
                               ybufs[j], gysem) for j in range(nbuf)]
        wx = []
        for j in range(nbuf):
            gx[j].wait()
            wx.append(pltpu.async_copy(
                xbufs[j], xo.at[pl.ds(base + j * ch, ch)], wsems[j]))

        for j in range(nbuf):
            gy[j].wait()
            xb, yb = xbufs[j], ybufs[j]

            def row_body(r, xb=xb, yb=yb, j=j):
                acc = xb[r, pl.ds(0, _LANES)] * yb[r, pl.ds(0, _LANES)]
                for k in range(1, _EMB // _LANES):
                    acc = acc + (xb[r, pl.ds(k * _LANES, _LANES)] *
                                 yb[r, pl.ds(k * _LANES, _LANES)])
                pos_v[j * ch + r, pl.ds(0, _LANES)] = acc

            plsc.parallel_loop(0, ch, 1, unroll=4)(row_body)
        wp = pltpu.async_copy(pos_v, po.at[pl.ds(base, bpw)], psem)

        if with_neg:
            @pl.when(wid == 0)
            def _():
                pltpu.sync_copy(ni, nidx_v)
                pltpu.async_copy(w2h.at[nidx_v], nrows_v, isem).wait()
                pltpu.sync_copy(nrows_v, no)

        for j in range(nbuf):
            wx[j].wait()
        wp.wait()

    args = (x_idx, y_idx) + ((neg_idx,) if with_neg else ()) + (w1, w2)
    return gather_kernel(*args)


def _tc_loss(x_emb, pos, neg_emb, prev, final):
    batch = x_emb.shape[0]
    blk = 2048
    nblk = batch // blk

    def logsig(z):
        return jnp.minimum(z, 0.0) - jnp.log1p(jnp.exp(-jnp.abs(z)))

    def body(neg_ref, prev_ref, x_ref, pos_ref, o_ref, acc_ref):
        i = pl.program_id(0)

        @pl.when(i == 0)
        def _():
            acc_ref[0] = prev_ref[0, 0]

        x = x_ref[...]
        scores = -lax.dot_general(
            x, neg_ref[...], (((1,), (1,)), ((), ())),
            preferred_element_type=jnp.float32)
        pos = jnp.sum(pos_ref[...], axis=1, keepdims=True)
        tot = jnp.sum(logsig(pos)) + jnp.sum(logsig(scores))
        acc_ref[0] = acc_ref[0] + tot

        @pl.when(i == nblk - 1)
        def _():
            if final:
                o_ref[0, 0] = -acc_ref[0] / _BATCH
            else:
                o_ref[0, 0] = acc_ref[0]

    return pl.pallas_call(
        body,
        grid=(nblk,),
        in_specs=[
            pl.BlockSpec((_NEG, _EMB), lambda i: (0, 0)),
            pl.BlockSpec(memory_space=pltpu.SMEM),
            pl.BlockSpec((blk, _EMB), lambda i: (i, 0)),
            pl.BlockSpec((blk, _LANES), lambda i: (i, 0)),
        ],
        out_specs=pl.BlockSpec(memory_space=pltpu.SMEM),
        out_shape=jax.ShapeDtypeStruct((1, 1), jnp.float32),
        scratch_shapes=[pltpu.SMEM((1,), jnp.float32)],
    )(neg_emb, prev, x_emb, pos)


def kernel(x, y, word_to_embedding, embedding_to_context, negative_samples):
    x = x.astype(jnp.int32)
    y = y.astype(jnp.int32)
    neg = negative_samples.astype(jnp.int32)
    w1, w2 = word_to_embedding, embedding_to_context

    x_emb0, pos0, neg_emb = _sc_gather(
        x[:_HALF], y[:_HALF], neg, w1, w2, 0, _HALF)
    x_emb1, pos1, _ = _sc_gather(
        x[_HALF:], y[_HALF:], neg, w1, w2, 0, _HALF)
    zero = jnp.zeros((1, 1), jnp.float32)
    p0 = _tc_loss(x_emb0, pos0, neg_emb, zero, final=False)
    loss = _tc_loss(x_emb1, pos1, neg_emb, p0, final=True)
    return loss.reshape(())


# speedup vs baseline: 1.0852x; 1.0852x over previous
import functools

import jax
import jax.numpy as jnp
from jax import lax
from jax.experimental import pallas as pl
from jax.experimental.pallas import tpu as pltpu
from jax.experimental.pallas import tpu_sc as plsc

_VOCAB = 100000
_EMB = 128
_BATCH = 16384
_HALF = _BATCH // 2
_NEG = 20


def _sc_gather(x_idx, y_idx, neg_idx, w1, w2, base_off, batch):
    info = plsc.get_sparse_core_info()
    nc, ns = info.num_cores, info.num_subcores
    nw = nc * ns
    bpw = batch // nw
    nbuf = 4
    ch = bpw // nbuf

    out_type = [
        jax.ShapeDtypeStruct((batch, _EMB), jnp.float32),
        jax.ShapeDtypeStruct((batch, _EMB), jnp.float32),
    ]
    with_neg = neg_idx is not None
    if with_neg:
        out_type.append(jax.ShapeDtypeStruct((_NEG, _EMB), jnp.float32))

    scratch = [
        pltpu.VMEM((bpw,), jnp.int32),
        pltpu.VMEM((bpw,), jnp.int32),
        [pltpu.VMEM((ch, _EMB), jnp.float32)] * nbuf,
        pltpu.SemaphoreType.DMA,
        [pltpu.SemaphoreType.DMA] * nbuf,
        pltpu.SemaphoreType.DMA,
    ]
    if with_neg:
        scratch += [pltpu.VMEM((_NEG,), jnp.int32),
                    pltpu.VMEM((_NEG, _EMB), jnp.float32)]

    @functools.partial(
        pl.kernel,
        out_type=tuple(out_type),
        mesh=plsc.VectorSubcoreMesh(core_axis_name="c", subcore_axis_name="s"),
        scratch_types=scratch,
    )
    def gather_kernel(*refs):
        if with_neg:
            (xi, yi, ni, w1h, w2h, xo, yo, no,
             idx_x, idx_y, bufs, gsem, wsems, isem, nidx_v, nrows_v) = refs
        else:
            (xi, yi, w1h, w2h, xo, yo,
             idx_x, idx_y, bufs, gsem, wsems, isem) = refs
        wid = lax.axis_index("s") * nc + lax.axis_index("c")
        base = wid * bpw
        ix = pltpu.async_copy(xi.at[pl.ds(base_off + base, bpw)], idx_x, isem)
        iy = pltpu.async_copy(yi.at[pl.ds(base_off + base, bpw)], idx_y, isem)
        ix.wait()
        gx = [pltpu.async_copy(w1h.at[idx_x.at[pl.ds(j * ch, ch)]],
                               bufs[j], gsem) for j in range(nbuf)]
        iy.wait()
        wx = []
        for j in range(nbuf):
            gx[j].wait()
            wx.append(pltpu.async_copy(
                bufs[j], xo.at[pl.ds(base + j * ch, ch)], wsems[j]))
        gy = []
        for j in range(nbuf):
            wx[j].wait()
            gy.append(pltpu.async_copy(w2h.at[idx_y.at[pl.ds(j * ch, ch)]],
                                       bufs[j], gsem))
        wy = []
        for j in range(nbuf):
            gy[j].wait()
            wy.append(pltpu.async_copy(
                bufs[j], yo.at[pl.ds(base + j * ch, ch)], wsems[j]))

        if with_neg:
            @pl.when(wid == 0)
            def _():
                pltpu.sync_copy(ni, nidx_v)
                pltpu.async_copy(w2h.at[nidx_v], nrows_v, isem).wait()
                pltpu.sync_copy(nrows_v, no)

        for j in range(nbuf):
            wy[j].wait()

    args = (x_idx, y_idx) + ((neg_idx,) if with_neg else ()) + (w1, w2)
    return gather_kernel(*args)


def _tc_loss(x_emb, y_emb, neg_emb, prev, final):
    batch = x_emb.shape[0]
    blk = 2048
    nblk = batch // blk

    def logsig(z):
        return jnp.minimum(z, 0.0) - jnp.log1p(jnp.exp(-jnp.abs(z)))

    def body(neg_ref, prev_ref, x_ref, y_ref, o_ref, acc_ref):
        i = pl.program_id(0)

        @pl.when(i == 0)
        def _():
            acc_ref[0] = prev_ref[0, 0]

        x = x_ref[...]
        y = y_ref[...]
        ones = jnp.ones((_EMB, 1), jnp.float32)
        pos = lax.dot_general(x * y, ones, (((1,), (0,)), ((), ())),
                              preferred_element_type=jnp.float32)
        scores = -lax.dot_general(
            x, neg_ref[...], (((1,), (1,)), ((), ())),
            preferred_element_type=jnp.float32)
        tot = jnp.sum(logsig(pos)) + jnp.sum(logsig(scores))
        acc_ref[0] = acc_ref[0] + tot

        @pl.when(i == nblk - 1)
        def _():
            if final:
                o_ref[0, 0] = -acc_ref[0] / _BATCH
            else:
                o_ref[0, 0] = acc_ref[0]

    return pl.pallas_call(
        body,
        grid=(nblk,),
        in_specs=[
            pl.BlockSpec((_NEG, _EMB), lambda i: (0, 0)),
            pl.BlockSpec(memory_space=pltpu.SMEM),
            pl.BlockSpec((blk, _EMB), lambda i: (i, 0)),
            pl.BlockSpec((blk, _EMB), lambda i: (i, 0)),
        ],
        out_specs=pl.BlockSpec(memory_space=pltpu.SMEM),
        out_shape=jax.ShapeDtypeStruct((1, 1), jnp.float32),
        scratch_shapes=[pltpu.SMEM((1,), jnp.float32)],
    )(neg_emb, prev, x_emb, y_emb)


def kernel(x, y, word_to_embedding, embedding_to_context, negative_samples):
    x = x.astype(jnp.int32)
    y = y.astype(jnp.int32)
    neg = negative_samples.astype(jnp.int32)
    w1, w2 = word_to_embedding, embedding_to_context

    x_emb0, y_emb0, neg_emb = _sc_gather(x, y, neg, w1, w2, 0, _HALF)
    x_emb1, y_emb1 = _sc_gather(x, y, None, w1, w2, _HALF, _HALF)
    zero = jnp.zeros((1, 1), jnp.float32)
    p0 = _tc_loss(x_emb0, y_emb0, neg_emb, zero, final=False)
    loss = _tc_loss(x_emb1, y_emb1, neg_emb, p0, final=True)
    return loss.reshape(())


# speedup vs baseline: 1.0995x; 1.0131x over previous
import functools

import jax
import jax.numpy as jnp
from jax import lax
from jax.experimental import pallas as pl
from jax.experimental.pallas import tpu as pltpu
from jax.experimental.pallas import tpu_sc as plsc

_VOCAB = 100000
_EMB = 128
_BATCH = 16384
_HALF = _BATCH // 2
_NEG = 20


def _sc_gather(x_idx, y_idx, neg_idx, w1, w2, base_off, batch):
    info = plsc.get_sparse_core_info()
    nc, ns = info.num_cores, info.num_subcores
    nw = nc * ns
    bpw = batch // nw
    nbuf = 2
    ch = bpw // nbuf

    out_type = [
        jax.ShapeDtypeStruct((batch, _EMB), jnp.float32),
        jax.ShapeDtypeStruct((batch, _EMB), jnp.float32),
    ]
    with_neg = neg_idx is not None
    if with_neg:
        out_type.append(jax.ShapeDtypeStruct((_NEG, _EMB), jnp.float32))

    scratch = [
        pltpu.VMEM((bpw,), jnp.int32),
        pltpu.VMEM((bpw,), jnp.int32),
        [pltpu.VMEM((ch, _EMB), jnp.float32)] * nbuf,
        pltpu.SemaphoreType.DMA,
        [pltpu.SemaphoreType.DMA] * nbuf,
        pltpu.SemaphoreType.DMA,
    ]
    if with_neg:
        scratch += [pltpu.VMEM((_NEG,), jnp.int32),
                    pltpu.VMEM((_NEG, _EMB), jnp.float32)]

    @functools.partial(
        pl.kernel,
        out_type=tuple(out_type),
        mesh=plsc.VectorSubcoreMesh(core_axis_name="c", subcore_axis_name="s"),
        scratch_types=scratch,
    )
    def gather_kernel(*refs):
        if with_neg:
            (xi, yi, ni, w1h, w2h, xo, yo, no,
             idx_x, idx_y, bufs, gsem, wsems, isem, nidx_v, nrows_v) = refs
        else:
            (xi, yi, w1h, w2h, xo, yo,
             idx_x, idx_y, bufs, gsem, wsems, isem) = refs
        wid = lax.axis_index("s") * nc + lax.axis_index("c")
        base = wid * bpw
        ix = pltpu.async_copy(xi.at[pl.ds(base_off + base, bpw)], idx_x, isem)
        iy = pltpu.async_copy(yi.at[pl.ds(base_off + base, bpw)], idx_y, isem)
        ix.wait()
        gx = [pltpu.async_copy(w1h.at[idx_x.at[pl.ds(j * ch, ch)]],
                               bufs[j], gsem) for j in range(nbuf)]
        iy.wait()
        wx = []
        for j in range(nbuf):
            gx[j].wait()
            wx.append(pltpu.async_copy(
                bufs[j], xo.at[pl.ds(base + j * ch, ch)], wsems[j]))
        gy = []
        for j in range(nbuf):
            wx[j].wait()
            gy.append(pltpu.async_copy(w2h.at[idx_y.at[pl.ds(j * ch, ch)]],
                                       bufs[j], gsem))
        wy = []
        for j in range(nbuf):
            gy[j].wait()
            wy.append(pltpu.async_copy(
                bufs[j], yo.at[pl.ds(base + j * ch, ch)], wsems[j]))

        if with_neg:
            @pl.when(wid == 0)
            def _():
                pltpu.sync_copy(ni, nidx_v)
                pltpu.async_copy(w2h.at[nidx_v], nrows_v, isem).wait()
                pltpu.sync_copy(nrows_v, no)

        for j in range(nbuf):
            wy[j].wait()

    args = (x_idx, y_idx) + ((neg_idx,) if with_neg else ()) + (w1, w2)
    return gather_kernel(*args)


def _tc_loss(x_emb, y_emb, neg_emb, prev, final):
    batch = x_emb.shape[0]
    blk = 2048
    nblk = batch // blk

    def logsig(z):
        return jnp.minimum(z, 0.0) - jnp.log1p(jnp.exp(-jnp.abs(z)))

    def body(neg_ref, prev_ref, x_ref, y_ref, o_ref, acc_ref):
        i = pl.program_id(0)

        @pl.when(i == 0)
        def _():
            acc_ref[0] = prev_ref[0, 0]

        x = x_ref[...]
        y = y_ref[...]
        ones = jnp.ones((_EMB, 1), jnp.float32)
        pos = lax.dot_general(x * y, ones, (((1,), (0,)), ((), ())),
                              preferred_element_type=jnp.float32)
        scores = -lax.dot_general(
            x, neg_ref[...], (((1,), (1,)), ((), ())),
            preferred_element_type=jnp.float32)
        tot = jnp.sum(logsig(pos)) + jnp.sum(logsig(scores))
        acc_ref[0] = acc_ref[0] + tot

        @pl.when(i == nblk - 1)
        def _():
            if final:
                o_ref[0, 0] = -acc_ref[0] / _BATCH
            else:
                o_ref[0, 0] = acc_ref[0]

    return pl.pallas_call(
        body,
        grid=(nblk,),
        in_specs=[
            pl.BlockSpec((_NEG, _EMB), lambda i: (0, 0)),
            pl.BlockSpec(memory_space=pltpu.SMEM),
            pl.BlockSpec((blk, _EMB), lambda i: (i, 0)),
            pl.BlockSpec((blk, _EMB), lambda i: (i, 0)),
        ],
        out_specs=pl.BlockSpec(memory_space=pltpu.SMEM),
        out_shape=jax.ShapeDtypeStruct((1, 1), jnp.float32),
        scratch_shapes=[pltpu.SMEM((1,), jnp.float32)],
    )(neg_emb, prev, x_emb, y_emb)


def kernel(x, y, word_to_embedding, embedding_to_context, negative_samples):
    x = x.astype(jnp.int32)
    y = y.astype(jnp.int32)
    neg = negative_samples.astype(jnp.int32)
    w1, w2 = word_to_embedding, embedding_to_context

    x_emb0, y_emb0, neg_emb = _sc_gather(x, y, neg, w1, w2, 0, _HALF)
    x_emb1, y_emb1 = _sc_gather(x, y, None, w1, w2, _HALF, _HALF)
    zero = jnp.zeros((1, 1), jnp.float32)
    p0 = _tc_loss(x_emb0, y_emb0, neg_emb, zero, final=False)
    loss = _tc_loss(x_emb1, y_emb1, neg_emb, p0, final=True)
    return loss.reshape(())


# speedup vs baseline: 1.1100x; 1.0095x over previous
import functools

import jax
import jax.numpy as jnp
from jax import lax
from jax.experimental import pallas as pl
from jax.experimental.pallas import tpu as pltpu
from jax.experimental.pallas import tpu_sc as plsc

_VOCAB = 100000
_EMB = 128
_BATCH = 16384
_HALF = _BATCH // 2
_NEG = 20


def _sc_gather(x_idx, y_idx, neg_idx, w1, w2, base_off, batch):
    info = plsc.get_sparse_core_info()
    nc, ns = info.num_cores, info.num_subcores
    nw = nc * ns
    bpw = batch // nw
    nbuf = 2
    ch = bpw // nbuf

    out_type = [
        jax.ShapeDtypeStruct((batch, _EMB), jnp.float32),
        jax.ShapeDtypeStruct((batch, _EMB), jnp.float32),
    ]
    with_neg = neg_idx is not None
    if with_neg:
        out_type.append(jax.ShapeDtypeStruct((_NEG, _EMB), jnp.float32))

    scratch = [
        pltpu.VMEM((bpw,), jnp.int32),
        pltpu.VMEM((bpw,), jnp.int32),
        [pltpu.VMEM((ch, _EMB), jnp.float32)] * nbuf,
        [pltpu.VMEM((ch, _EMB), jnp.float32)] * nbuf,
        pltpu.SemaphoreType.DMA,
        [pltpu.SemaphoreType.DMA] * nbuf,
        [pltpu.SemaphoreType.DMA] * nbuf,
        pltpu.SemaphoreType.DMA,
    ]
    if with_neg:
        scratch += [pltpu.VMEM((_NEG,), jnp.int32),
                    pltpu.VMEM((_NEG, _EMB), jnp.float32)]

    @functools.partial(
        pl.kernel,
        out_type=tuple(out_type),
        mesh=plsc.VectorSubcoreMesh(core_axis_name="c", subcore_axis_name="s"),
        scratch_types=scratch,
    )
    def gather_kernel(*refs):
        if with_neg:
            (xi, yi, ni, w1h, w2h, xo, yo, no,
             idx_x, idx_y, xbufs, ybufs, gsem, wxsems, wysems, isem,
             nidx_v, nrows_v) = refs
        else:
            (xi, yi, w1h, w2h, xo, yo,
             idx_x, idx_y, xbufs, ybufs, gsem, wxsems, wysems, isem) = refs
        wid = lax.axis_index("s") * nc + lax.axis_index("c")
        base = wid * bpw
        ix = pltpu.async_copy(xi.at[pl.ds(base_off + base, bpw)], idx_x, isem)
        iy = pltpu.async_copy(yi.at[pl.ds(base_off + base, bpw)], idx_y, isem)
        ix.wait()
        gx = [pltpu.async_copy(w1h.at[idx_x.at[pl.ds(j * ch, ch)]],
                               xbufs[j], gsem) for j in range(nbuf)]
        iy.wait()
        gy = [pltpu.async_copy(w2h.at[idx_y.at[pl.ds(j * ch, ch)]],
                               ybufs[j], gsem) for j in range(nbuf)]
        wx, wy = [], []
        for j in range(nbuf):
            gx[j].wait()
            wx.append(pltpu.async_copy(
                xbufs[j], xo.at[pl.ds(base + j * ch, ch)], wxsems[j]))
        for j in range(nbuf):
            gy[j].wait()
            wy.append(pltpu.async_copy(
                ybufs[j], yo.at[pl.ds(base + j * ch, ch)], wysems[j]))

        if with_neg:
            @pl.when(wid == 0)
            def _():
                pltpu.sync_copy(ni, nidx_v)
                pltpu.async_copy(w2h.at[nidx_v], nrows_v, isem).wait()
                pltpu.sync_copy(nrows_v, no)

        for j in range(nbuf):
            wx[j].wait()
            wy[j].wait()

    args = (x_idx, y_idx) + ((neg_idx,) if with_neg else ()) + (w1, w2)
    return gather_kernel(*args)


def _tc_loss(x_emb, y_emb, neg_emb, prev, final):
    batch = x_emb.shape[0]
    blk = 2048
    nblk = batch // blk

    def logsig(z):
        return jnp.minimum(z, 0.0) - jnp.log1p(jnp.exp(-jnp.abs(z)))

    def body(neg_ref, prev_ref, x_ref, y_ref, o_ref, acc_ref):
        i = pl.program_id(0)

        @pl.when(i == 0)
        def _():
            acc_ref[0] = prev_ref[0, 0]

        x = x_ref[...]
        y = y_ref[...]
        ones = jnp.ones((_EMB, 1), jnp.float32)
        pos = lax.dot_general(x * y, ones, (((1,), (0,)), ((), ())),
                              preferred_element_type=jnp.float32)
        scores = -lax.dot_general(
            x, neg_ref[...], (((1,), (1,)), ((), ())),
            preferred_element_type=jnp.float32)
        tot = jnp.sum(logsig(pos)) + jnp.sum(logsig(scores))
        acc_ref[0] = acc_ref[0] + tot

        @pl.when(i == nblk - 1)
        def _():
            if final:
                o_ref[0, 0] = -acc_ref[0] / _BATCH
            else:
                o_ref[0, 0] = acc_ref[0]

    return pl.pallas_call(
        body,
        grid=(nblk,),
        in_specs=[
            pl.BlockSpec((_NEG, _EMB), lambda i: (0, 0)),
            pl.BlockSpec(memory_space=pltpu.SMEM),
            pl.BlockSpec((blk, _EMB), lambda i: (i, 0)),
            pl.BlockSpec((blk, _EMB), lambda i: (i, 0)),
        ],
        out_specs=pl.BlockSpec(memory_space=pltpu.SMEM),
        out_shape=jax.ShapeDtypeStruct((1, 1), jnp.float32),
        scratch_shapes=[pltpu.SMEM((1,), jnp.float32)],
    )(neg_emb, prev, x_emb, y_emb)


def kernel(x, y, word_to_embedding, embedding_to_context, negative_samples):
    x = x.astype(jnp.int32)
    y = y.astype(jnp.int32)
    neg = negative_samples.astype(jnp.int32)
    w1, w2 = word_to_embedding, embedding_to_context

    x_emb0, y_emb0, neg_emb = _sc_gather(x, y, neg, w1, w2, 0, _HALF)
    x_emb1, y_emb1 = _sc_gather(x, y, None, w1, w2, _HALF, _HALF)
    zero = jnp.zeros((1, 1), jnp.float32)
    p0 = _tc_loss(x_emb0, y_emb0, neg_emb, zero, final=False)
    loss = _tc_loss(x_emb1, y_emb1, neg_emb, p0, final=True)
    return loss.reshape(())
